# Initial kernel scaffold; baseline (speedup 1.0000x reference)
import jax
import jax.numpy as jnp
from jax.experimental import pallas as pl


def kernel(scores):
    raise NotImplementedError("write your pallas kernel here")


# speedup vs baseline: 3.2105x; 3.2105x over previous
import functools

import jax
import jax.numpy as jnp
from jax import lax
from jax.experimental import pallas as pl
from jax.experimental.pallas import tpu as pltpu

_K = 10
_NSUB = 1 << _K
_BIGI = 2 ** 30


def _select_kernel(s_ref, idx_ref, mcode_ref, cost_ref):
    s = s_ref[...]
    B, N = s.shape
    a = jnp.abs(s)
    pos = lax.broadcasted_iota(jnp.int32, (B, N), 1)

    vals = []
    for i in range(_K):
        m = jnp.min(a, axis=1, keepdims=True)
        isel = jnp.min(jnp.where(a == m, pos, _BIGI), axis=1, keepdims=True)
        vals.append(m)
        idx_ref[:, i:i + 1] = isel
        a = jnp.where(pos == isel, jnp.float32(jnp.inf), a)
    for i in range(_K, 16):
        idx_ref[:, i:i + 1] = jnp.full((B, 1), -1, jnp.int32)

    mm = lax.broadcasted_iota(jnp.int32, (B, _NSUB), 1)
    sums = jnp.zeros((B, _NSUB), jnp.float32)
    for i in range(_K):
        bit = ((mm >> i) & 1) == 1
        sums = sums + jnp.where(bit, vals[i], jnp.float32(0.0))

    for c in range(_K):
        m = jnp.min(sums, axis=1, keepdims=True)
        msel = jnp.min(jnp.where(sums == m, mm, _BIGI), axis=1, keepdims=True)
        cost_ref[:, c:c + 1] = m
        mcode_ref[:, c:c + 1] = msel
        sums = jnp.where(mm == msel, jnp.float32(jnp.inf), sums)
    for c in range(_K, 16):
        cost_ref[:, c:c + 1] = jnp.zeros((B, 1), jnp.float32)
        mcode_ref[:, c:c + 1] = jnp.zeros((B, 1), jnp.int32)


def _materialize_kernel(idx_ref, mcode_ref, s_ref, z_ref):
    b = pl.program_id(0)
    s2 = s_ref[0]
    R, L = s2.shape
    basef = (s2 > 0).astype(jnp.float32)
    ri = lax.broadcasted_iota(jnp.int32, (R, L), 0)
    ci = lax.broadcasted_iota(jnp.int32, (R, L), 1)
    flat = ri * L + ci

    code = jnp.zeros((R, L), jnp.int32)
    for i in range(_K):
        p = idx_ref[b, i]
        w = jnp.int32(0)
        for c in range(_K):
            w = w | (((mcode_ref[b, c] >> i) & 1) << c)
        code = jnp.where(flat == p, w, code)

    one = jnp.float32(1.0)
    for c in range(_K):
        bit = ((code >> c) & 1) > 0
        z_ref[0, c] = jnp.where(bit, one - basef, basef)


def _sparsemax_kernel(sk_ref, distr_ref, ent_ref):
    sk = sk_ref[...]
    B, K = sk.shape
    lane = lax.broadcasted_iota(jnp.int32, (B, K), 1)

    rem = sk
    sorted_vals = []
    for r in range(K):
        m = jnp.max(rem, axis=1, keepdims=True)
        am = jnp.min(jnp.where(rem == m, lane, _BIGI), axis=1, keepdims=True)
        sorted_vals.append(m)
        rem = jnp.where(lane == am, jnp.float32(-jnp.inf), rem)

    cums = [sorted_vals[0]]
    for r in range(1, K):
        cums.append(cums[-1] + sorted_vals[r])

    k_z = jnp.zeros((B, 1), jnp.int32)
    for r in range(K):
        rf = jnp.float32(r + 1)
        support = (1.0 + rf * sorted_vals[r]) > cums[r]
        k_z = k_z + support.astype(jnp.int32)

    tau_num = jnp.zeros((B, 1), jnp.float32)
    for r in range(K):
        tau_num = jnp.where(k_z == r + 1, cums[r], tau_num)
    tau = (tau_num - 1.0) / k_z.astype(jnp.float32)

    distr = jnp.maximum(sk - tau, 0.0)
    distr_ref[...] = distr

    safe = jnp.where(distr > 0, distr, jnp.float32(1.0))
    plogp = jnp.where(distr > 0, distr * jnp.log(safe), jnp.float32(0.0))
    total = jnp.sum(plogp)
    ent_ref[...] = (-total / jnp.float32(B)).reshape(1, 1)


def kernel(scores):
    B, N = scores.shape
    R = N // 128

    idx, mcode, cost = pl.pallas_call(
        _select_kernel,
        in_specs=[pl.BlockSpec((B, N), lambda: (0, 0))],
        out_specs=[
            pl.BlockSpec((B, 16), lambda: (0, 0)),
            pl.BlockSpec((B, 16), lambda: (0, 0)),
            pl.BlockSpec((B, 16), lambda: (0, 0)),
        ],
        out_shape=[
            jax.ShapeDtypeStruct((B, 16), jnp.int32),
            jax.ShapeDtypeStruct((B, 16), jnp.int32),
            jax.ShapeDtypeStruct((B, 16), jnp.float32),
        ],
    )(scores)

    scores3 = scores.reshape(B, R, 128)
    z4 = pl.pallas_call(
        _materialize_kernel,
        grid=(B,),
        in_specs=[
            pl.BlockSpec(memory_space=pltpu.SMEM),
            pl.BlockSpec(memory_space=pltpu.SMEM),
            pl.BlockSpec((1, R, 128), lambda b: (b, 0, 0)),
        ],
        out_specs=pl.BlockSpec((1, _K, R, 128), lambda b: (b, 0, 0, 0)),
        out_shape=jax.ShapeDtypeStruct((B, _K, R, 128), jnp.float32),
    )(idx, mcode, scores3)
    bit_vector_z = z4.reshape(B, _K, N)

    s_k = jnp.einsum('bkj,bj->bk', bit_vector_z, scores)

    distr, ent = pl.pallas_call(
        _sparsemax_kernel,
        in_specs=[pl.BlockSpec((B, _K), lambda: (0, 0))],
        out_specs=[
            pl.BlockSpec((B, _K), lambda: (0, 0)),
            pl.BlockSpec((1, 1), lambda: (0, 0)),
        ],
        out_shape=[
            jax.ShapeDtypeStruct((B, _K), jnp.float32),
            jax.ShapeDtypeStruct((1, 1), jnp.float32),
        ],
    )(s_k)

    return bit_vector_z, distr, ent.reshape(())


# speedup vs baseline: 6.6271x; 2.0642x over previous
import functools

import jax
import jax.numpy as jnp
from jax import lax
from jax.experimental import pallas as pl
from jax.experimental.pallas import tpu as pltpu

_K = 10
_NSUB = 1 << _K
_BIGI = 2 ** 30


def _select_kernel(s_ref, idx_ref, mcode_ref, cost_ref):
    s = s_ref[...]
    B, N = s.shape
    a = jnp.abs(s)
    pos = lax.broadcasted_iota(jnp.int32, (B, N), 1)

    vals = []
    for i in range(_K):
        m = jnp.min(a, axis=1, keepdims=True)
        isel = jnp.min(jnp.where(a == m, pos, _BIGI), axis=1, keepdims=True)
        vals.append(m)
        idx_ref[:, i:i + 1] = isel
        a = jnp.where(pos == isel, jnp.float32(jnp.inf), a)
    for i in range(_K, 16):
        idx_ref[:, i:i + 1] = jnp.full((B, 1), -1, jnp.int32)

    mm = lax.broadcasted_iota(jnp.int32, (B, _NSUB), 1)
    sums = jnp.zeros((B, _NSUB), jnp.float32)
    for i in range(_K):
        bit = ((mm >> i) & 1) == 1
        sums = sums + jnp.where(bit, vals[i], jnp.float32(0.0))

    for c in range(_K):
        m = jnp.min(sums, axis=1, keepdims=True)
        msel = jnp.min(jnp.where(sums == m, mm, _BIGI), axis=1, keepdims=True)
        cost_ref[:, c:c + 1] = m
        mcode_ref[:, c:c + 1] = msel
        sums = jnp.where(mm == msel, jnp.float32(jnp.inf), sums)
    for c in range(_K, 16):
        cost_ref[:, c:c + 1] = jnp.zeros((B, 1), jnp.float32)
        mcode_ref[:, c:c + 1] = jnp.zeros((B, 1), jnp.int32)


def _materialize_kernel(idx_ref, mcode_ref, s_ref, z_ref):
    g = pl.program_id(0)
    s8 = s_ref[...]
    Bs, N = s8.shape
    base8 = (s8 > 0).astype(jnp.float32)
    nbase8 = 1.0 - base8
    lane = lax.broadcasted_iota(jnp.int32, (Bs, N), 1)
    rowi = lax.broadcasted_iota(jnp.int32, (Bs, 1), 0)

    code8 = jnp.zeros((Bs, N), jnp.int32)
    for i in range(_K):
        idxv = jnp.zeros((Bs, 1), jnp.int32)
        wv = jnp.zeros((Bs, 1), jnp.int32)
        for row in range(Bs):
            b = g * Bs + row
            p = idx_ref[b, i]
            w = jnp.int32(0)
            for c in range(_K):
                w = w | (((mcode_ref[b, c] >> i) & 1) << c)
            idxv = jnp.where(rowi == row, p, idxv)
            wv = jnp.where(rowi == row, w, wv)
        code8 = jnp.where(lane == idxv, wv, code8)

    for c in range(_K):
        bit = (code8 >> c) & 1
        z_ref[c] = jnp.where(bit == 1, nbase8, base8)


def _sparsemax_kernel(sk_ref, distr_ref, ent_ref):
    sk = sk_ref[...]
    B, K = sk.shape
    lane = lax.broadcasted_iota(jnp.int32, (B, K), 1)

    rem = sk
    sorted_vals = []
    for r in range(K):
        m = jnp.max(rem, axis=1, keepdims=True)
        am = jnp.min(jnp.where(rem == m, lane, _BIGI), axis=1, keepdims=True)
        sorted_vals.append(m)
        rem = jnp.where(lane == am, jnp.float32(-jnp.inf), rem)

    cums = [sorted_vals[0]]
    for r in range(1, K):
        cums.append(cums[-1] + sorted_vals[r])

    k_z = jnp.zeros((B, 1), jnp.int32)
    for r in range(K):
        rf = jnp.float32(r + 1)
        support = (1.0 + rf * sorted_vals[r]) > cums[r]
        k_z = k_z + support.astype(jnp.int32)

    tau_num = jnp.zeros((B, 1), jnp.float32)
    for r in range(K):
        tau_num = jnp.where(k_z == r + 1, cums[r], tau_num)
    tau = (tau_num - 1.0) / k_z.astype(jnp.float32)

    distr = jnp.maximum(sk - tau, 0.0)
    distr_ref[...] = distr

    safe = jnp.where(distr > 0, distr, jnp.float32(1.0))
    plogp = jnp.where(distr > 0, distr * jnp.log(safe), jnp.float32(0.0))
    total = jnp.sum(plogp)
    ent_ref[...] = (-total / jnp.float32(B)).reshape(1, 1)


def kernel(scores):
    B, N = scores.shape
    R = N // 128

    idx, mcode, cost = pl.pallas_call(
        _select_kernel,
        in_specs=[pl.BlockSpec((B, N), lambda: (0, 0))],
        out_specs=[
            pl.BlockSpec((B, 16), lambda: (0, 0)),
            pl.BlockSpec((B, 16), lambda: (0, 0)),
            pl.BlockSpec((B, 16), lambda: (0, 0)),
        ],
        out_shape=[
            jax.ShapeDtypeStruct((B, 16), jnp.int32),
            jax.ShapeDtypeStruct((B, 16), jnp.int32),
            jax.ShapeDtypeStruct((B, 16), jnp.float32),
        ],
    )(scores)

    GB = 8
    z_t = pl.pallas_call(
        _materialize_kernel,
        grid=(B // GB,),
        in_specs=[
            pl.BlockSpec(memory_space=pltpu.SMEM),
            pl.BlockSpec(memory_space=pltpu.SMEM),
            pl.BlockSpec((GB, N), lambda g: (g, 0)),
        ],
        out_specs=pl.BlockSpec((_K, GB, N), lambda g: (0, g, 0)),
        out_shape=jax.ShapeDtypeStruct((_K, B, N), jnp.float32),
    )(idx, mcode, scores)
    bit_vector_z = jnp.transpose(z_t, (1, 0, 2))

    s_k = jnp.einsum('bkj,bj->bk', bit_vector_z, scores)

    distr, ent = pl.pallas_call(
        _sparsemax_kernel,
        in_specs=[pl.BlockSpec((B, _K), lambda: (0, 0))],
        out_specs=[
            pl.BlockSpec((B, _K), lambda: (0, 0)),
            pl.BlockSpec((1, 1), lambda: (0, 0)),
        ],
        out_shape=[
            jax.ShapeDtypeStruct((B, _K), jnp.float32),
            jax.ShapeDtypeStruct((1, 1), jnp.float32),
        ],
    )(s_k)

    return bit_vector_z, distr, ent.reshape(())
